# baseline (speedup 1.0000x reference)
import jax
import jax.numpy as jnp
from jax.experimental import pallas as pl
from jax.experimental.pallas import tpu as pltpu

_B, _N, _C = 16, 300, 92
_B2 = _B // 2
_T = 50
_TP = 64
_G = 262144
_G_BLK = 8192


def _global_loss_body(cls_p, cls_t, reg_p, reg_t, out_ref):
    i = pl.program_id(0)

    @pl.when(i == 0)
    def _():
        out_ref[0] = 0.0
        out_ref[1] = 0.0

    s_cls = jnp.sum(jnp.abs(cls_p[...] - cls_t[...]))
    d = reg_p[...] - reg_t[...]
    s_reg = jnp.sum(d * d)
    out_ref[0] += s_cls
    out_ref[1] += s_reg


def _half_reduce_min(x):
    lo = jnp.min(x[:, :, :64], axis=2, keepdims=True)
    hi = jnp.min(x[:, :, 64:], axis=2, keepdims=True)
    return jnp.concatenate(
        [jnp.broadcast_to(lo, (_B2, 1, 64)),
         jnp.broadcast_to(hi, (_B2, 1, 64))], axis=2)


def _match_body(coords_ref, logits_ref, tgt_ref, labels_ref, out_ref,
                cm_ref, cbb_ref, cgi_ref, rs_ref, m_ref):
    logits = logits_ref[...]
    rowsum = jnp.sum(logits, axis=2, keepdims=True)
    prob = jax.nn.softmax(logits, axis=-1)

    valid = jax.lax.broadcasted_iota(jnp.int32, (_N, _TP), 1) < _T

    for b in range(_B):
        b2, off = b // 2, (b % 2) * _TP
        prob_b = prob[b]
        lab = labels_ref[b]
        oh = (lab == jax.lax.broadcasted_iota(jnp.int32, (_C, _TP), 0))
        cclass = jax.lax.dot(prob_b, oh.astype(jnp.float32),
                             precision=jax.lax.Precision.HIGHEST)

        cb = coords_ref[b]
        cx, cy, w, h = cb[:, 0:1], cb[:, 1:2], cb[:, 2:3], cb[:, 3:4]
        tg = tgt_ref[b]
        tcx, tcy, tw, th = tg[0:1, :], tg[1:2, :], tg[2:3, :], tg[3:4, :]

        cbbox = (jnp.abs(cx - tcx) + jnp.abs(cy - tcy)
                 + jnp.abs(w - tw) + jnp.abs(h - th))

        x0, y0 = cx - 0.5 * w, cy - 0.5 * h
        x1, y1 = cx + 0.5 * w, cy + 0.5 * h
        tx0, ty0 = tcx - 0.5 * tw, tcy - 0.5 * th
        tx1, ty1 = tcx + 0.5 * tw, tcy + 0.5 * th
        area1 = (x1 - x0) * (y1 - y0)
        area2 = (tx1 - tx0) * (ty1 - ty0)
        inter = (jnp.clip(jnp.minimum(x1, tx1) - jnp.maximum(x0, tx0), 0.0)
                 * jnp.clip(jnp.minimum(y1, ty1) - jnp.maximum(y0, ty0), 0.0))
        union = area1 + area2 - inter
        iou = inter / union
        areae = (jnp.clip(jnp.maximum(x1, tx1) - jnp.minimum(x0, tx0), 0.0)
                 * jnp.clip(jnp.maximum(y1, ty1) - jnp.minimum(y0, ty0), 0.0))
        giou = iou - (areae - union) / areae

        cm = 5.0 * cbbox - cclass - 2.0 * giou
        cm_ref[b2, :, off:off + _TP] = jnp.where(valid, cm, jnp.inf)
        cbb_ref[b2, :, off:off + _TP] = cbbox
        cgi_ref[b2, :, off:off + _TP] = giou
        rs_ref[b2, :, off:off + _TP] = jnp.broadcast_to(rowsum[b], (_N, _TP))

    m_ref[...] = jnp.zeros_like(m_ref)
    iota_i = jax.lax.broadcasted_iota(jnp.int32, (_B2, _N, 2 * _TP), 1)
    iota_l = jax.lax.broadcasted_iota(jnp.int32, (_B2, _N, 2 * _TP), 2)
    flat = iota_i * (2 * _TP) + iota_l
    big = jnp.int32(2 ** 30)

    def body(_, carry):
        cmv = cm_ref[...]
        rowmin = jnp.min(cmv, axis=1, keepdims=True)
        bmin = _half_reduce_min(rowmin)
        cand = jnp.where(cmv == bmin, flat, big)
        frow = jnp.min(cand, axis=1, keepdims=True)
        fsel = _half_reduce_min(frow)
        i_b = fsel // (2 * _TP)
        j_l = fsel % (2 * _TP)
        kill = (iota_i == i_b) | (iota_l == j_l)
        m_ref[...] += (flat == fsel).astype(jnp.float32)
        cm_ref[...] = jnp.where(kill, jnp.inf, cmv)
        return carry

    jax.lax.fori_loop(0, _T, body, 0)

    m = m_ref[...]
    out_ref[0] = jnp.sum(m * cbb_ref[...])
    out_ref[1] = jnp.sum(m * cgi_ref[...])
    out_ref[2] = jnp.sum(m * rs_ref[...])


def kernel(box_coords, box_logits, tgt_boxes, g_cls_pred, g_cls_tgt,
           g_regr_pred, g_regr_tgt, tgt_labels):
    nblk = _G // _G_BLK
    gsums = pl.pallas_call(
        _global_loss_body,
        grid=(nblk,),
        in_specs=[pl.BlockSpec((_B, _G_BLK), lambda i: (0, i))] * 4,
        out_specs=pl.BlockSpec(memory_space=pltpu.SMEM),
        out_shape=jax.ShapeDtypeStruct((2,), jnp.float32),
    )(g_cls_pred, g_cls_tgt, g_regr_pred, g_regr_tgt)

    tgt_t = jnp.transpose(tgt_boxes, (0, 2, 1))
    tgt_t = jnp.pad(tgt_t, ((0, 0), (0, 0), (0, _TP - _T)))
    labels = jnp.pad(tgt_labels.astype(jnp.int32),
                     ((0, 0), (0, _TP - _T)),
                     constant_values=-1)[:, None, :]

    msums = pl.pallas_call(
        _match_body,
        out_specs=pl.BlockSpec(memory_space=pltpu.SMEM),
        out_shape=jax.ShapeDtypeStruct((3,), jnp.float32),
        scratch_shapes=[
            pltpu.VMEM((_B2, _N, 2 * _TP), jnp.float32),
            pltpu.VMEM((_B2, _N, 2 * _TP), jnp.float32),
            pltpu.VMEM((_B2, _N, 2 * _TP), jnp.float32),
            pltpu.VMEM((_B2, _N, 2 * _TP), jnp.float32),
            pltpu.VMEM((_B2, _N, 2 * _TP), jnp.float32),
        ],
    )(box_coords, box_logits, tgt_t, labels)

    denom = jnp.float32(_B * _G)
    num_boxes = jnp.float32(4.0 * _B)
    g_cls_loss = gsums[0] / denom
    g_regr_loss = gsums[1] / denom
    loss_bbox = msums[0] / num_boxes
    loss_giou = (jnp.float32(_B * _T) - msums[1]) / num_boxes
    loss_cls = -msums[2]
    return jnp.stack([g_cls_loss, g_regr_loss, loss_bbox, loss_giou,
                      loss_cls])


# speedup vs baseline: 1.1972x; 1.1972x over previous
import jax
import jax.numpy as jnp
from jax.experimental import pallas as pl
from jax.experimental.pallas import tpu as pltpu

_B, _N, _C = 16, 300, 92
_B2 = _B // 2
_T = 50
_TP = 64
_G = 262144
_G_BLK = 8192
_NBLK = _G // _G_BLK
_ITERS_PER_STEP = 2
_LOOP_STEPS = _T // _ITERS_PER_STEP


def _half_reduce_min(x):
    lo = jnp.min(x[:, :, :64], axis=2, keepdims=True)
    hi = jnp.min(x[:, :, 64:], axis=2, keepdims=True)
    return jnp.concatenate(
        [jnp.broadcast_to(lo, (_B2, 1, 64)),
         jnp.broadcast_to(hi, (_B2, 1, 64))], axis=2)


def _build_costs(coords_ref, logits_ref, tgt_ref, labels_ref,
                 cm_ref, cbb_ref, cgi_ref, rs_ref):
    logits = logits_ref[...]
    rowsum = jnp.sum(logits, axis=2, keepdims=True)
    prob = jax.nn.softmax(logits, axis=-1)

    valid = jax.lax.broadcasted_iota(jnp.int32, (_N, _TP), 1) < _T

    for b in range(_B):
        b2, off = b // 2, (b % 2) * _TP
        prob_b = prob[b]
        lab = labels_ref[b]
        oh = (lab == jax.lax.broadcasted_iota(jnp.int32, (_C, _TP), 0))
        cclass = jax.lax.dot(prob_b, oh.astype(jnp.float32),
                             precision=jax.lax.Precision.HIGHEST)

        cb = coords_ref[b]
        cx, cy, w, h = cb[:, 0:1], cb[:, 1:2], cb[:, 2:3], cb[:, 3:4]
        tg = tgt_ref[b]
        tcx, tcy, tw, th = tg[0:1, :], tg[1:2, :], tg[2:3, :], tg[3:4, :]

        cbbox = (jnp.abs(cx - tcx) + jnp.abs(cy - tcy)
                 + jnp.abs(w - tw) + jnp.abs(h - th))

        x0, y0 = cx - 0.5 * w, cy - 0.5 * h
        x1, y1 = cx + 0.5 * w, cy + 0.5 * h
        tx0, ty0 = tcx - 0.5 * tw, tcy - 0.5 * th
        tx1, ty1 = tcx + 0.5 * tw, tcy + 0.5 * th
        area1 = (x1 - x0) * (y1 - y0)
        area2 = (tx1 - tx0) * (ty1 - ty0)
        inter = (jnp.clip(jnp.minimum(x1, tx1) - jnp.maximum(x0, tx0), 0.0)
                 * jnp.clip(jnp.minimum(y1, ty1) - jnp.maximum(y0, ty0), 0.0))
        union = area1 + area2 - inter
        iou = inter / union
        areae = (jnp.clip(jnp.maximum(x1, tx1) - jnp.minimum(x0, tx0), 0.0)
                 * jnp.clip(jnp.maximum(y1, ty1) - jnp.minimum(y0, ty0), 0.0))
        giou = iou - (areae - union) / areae

        cm = 5.0 * cbbox - cclass - 2.0 * giou
        cm_ref[b2, :, off:off + _TP] = jnp.where(valid, cm, jnp.inf)
        cbb_ref[b2, :, off:off + _TP] = cbbox
        cgi_ref[b2, :, off:off + _TP] = giou
        rs_ref[b2, :, off:off + _TP] = jnp.broadcast_to(rowsum[b], (_N, _TP))


def _greedy_iter(cm_ref, m_ref, iota_i, iota_l, flat):
    big = jnp.int32(2 ** 30)
    cmv = cm_ref[...]
    rowmin = jnp.min(cmv, axis=1, keepdims=True)
    bmin = _half_reduce_min(rowmin)
    cand = jnp.where(cmv == bmin, flat, big)
    frow = jnp.min(cand, axis=1, keepdims=True)
    fsel = _half_reduce_min(frow)
    i_b = fsel // (2 * _TP)
    j_l = fsel % (2 * _TP)
    kill = (iota_i == i_b) | (iota_l == j_l)
    m_ref[...] += (flat == fsel).astype(jnp.float32)
    cm_ref[...] = jnp.where(kill, jnp.inf, cmv)


def _fused_body(coords_ref, logits_ref, tgt_ref, labels_ref,
                cls_p, cls_t, reg_p, reg_t, out_ref,
                cm_ref, cbb_ref, cgi_ref, rs_ref, m_ref):
    s = pl.program_id(0)

    @pl.when(s == 0)
    def _():
        for k in range(5):
            out_ref[k] = 0.0
        m_ref[...] = jnp.zeros_like(m_ref)
        _build_costs(coords_ref, logits_ref, tgt_ref, labels_ref,
                     cm_ref, cbb_ref, cgi_ref, rs_ref)

    s_cls = jnp.sum(jnp.abs(cls_p[...] - cls_t[...]))
    d = reg_p[...] - reg_t[...]
    out_ref[0] += s_cls
    out_ref[1] += jnp.sum(d * d)

    @pl.when((s >= 1) & (s <= _LOOP_STEPS))
    def _():
        iota_i = jax.lax.broadcasted_iota(jnp.int32, (_B2, _N, 2 * _TP), 1)
        iota_l = jax.lax.broadcasted_iota(jnp.int32, (_B2, _N, 2 * _TP), 2)
        flat = iota_i * (2 * _TP) + iota_l
        for _ in range(_ITERS_PER_STEP):
            _greedy_iter(cm_ref, m_ref, iota_i, iota_l, flat)

    @pl.when(s == _NBLK - 1)
    def _():
        m = m_ref[...]
        out_ref[2] = jnp.sum(m * cbb_ref[...])
        out_ref[3] = jnp.sum(m * cgi_ref[...])
        out_ref[4] = jnp.sum(m * rs_ref[...])


def kernel(box_coords, box_logits, tgt_boxes, g_cls_pred, g_cls_tgt,
           g_regr_pred, g_regr_tgt, tgt_labels):
    tgt_t = jnp.transpose(tgt_boxes, (0, 2, 1))
    tgt_t = jnp.pad(tgt_t, ((0, 0), (0, 0), (0, _TP - _T)))
    labels = jnp.pad(tgt_labels.astype(jnp.int32),
                     ((0, 0), (0, _TP - _T)),
                     constant_values=-1)[:, None, :]

    const3 = lambda i: (0, 0, 0)
    sums = pl.pallas_call(
        _fused_body,
        grid=(_NBLK,),
        in_specs=[
            pl.BlockSpec((_B, _N, 4), const3),
            pl.BlockSpec((_B, _N, _C), const3),
            pl.BlockSpec((_B, 4, _TP), const3),
            pl.BlockSpec((_B, 1, _TP), const3),
            pl.BlockSpec((_B, _G_BLK), lambda i: (0, i)),
            pl.BlockSpec((_B, _G_BLK), lambda i: (0, i)),
            pl.BlockSpec((_B, _G_BLK), lambda i: (0, i)),
            pl.BlockSpec((_B, _G_BLK), lambda i: (0, i)),
        ],
        out_specs=pl.BlockSpec(memory_space=pltpu.SMEM),
        out_shape=jax.ShapeDtypeStruct((5,), jnp.float32),
        scratch_shapes=[
            pltpu.VMEM((_B2, _N, 2 * _TP), jnp.float32),
            pltpu.VMEM((_B2, _N, 2 * _TP), jnp.float32),
            pltpu.VMEM((_B2, _N, 2 * _TP), jnp.float32),
            pltpu.VMEM((_B2, _N, 2 * _TP), jnp.float32),
            pltpu.VMEM((_B2, _N, 2 * _TP), jnp.float32),
        ],
    )(box_coords, box_logits, tgt_t, labels,
      g_cls_pred, g_cls_tgt, g_regr_pred, g_regr_tgt)

    denom = jnp.float32(_B * _G)
    num_boxes = jnp.float32(4.0 * _B)
    g_cls_loss = sums[0] / denom
    g_regr_loss = sums[1] / denom
    loss_bbox = sums[2] / num_boxes
    loss_giou = (jnp.float32(_B * _T) - sums[3]) / num_boxes
    loss_cls = -sums[4]
    return jnp.stack([g_cls_loss, g_regr_loss, loss_bbox, loss_giou,
                      loss_cls])
